# baseline (speedup 1.0000x reference)
import functools

import jax
import jax.numpy as jnp
from jax import lax
from jax.experimental import pallas as pl
from jax.experimental.pallas import tpu as pltpu
from jax.experimental.pallas import tpu_sc as plsc

M_ITEMS = 1_000_000
DIM = 16
LANES = 16
NUM_CORES = 2
NUM_SUBCORES = 16
NUM_WORKERS = 32

CHUNK = 800
GROUPS = CHUNK // LANES
KSTREAMS = 8
SUBW = CHUNK * DIM // KSTREAMS

NSC_CHUNKS = 66
N_SC = NSC_CHUNKS * CHUNK
N_TC = M_ITEMS - N_SC
TC_GRID = 37
RC = N_TC // 8 // TC_GRID
BK = RC * 128
assert RC * TC_GRID * 8 == N_TC and BK % 1024 == 0


def _tc_body(u_ref, x_ref, o_ref):
    xw = x_ref[...] * u_ref[...]
    j = lax.broadcasted_iota(jnp.int32, (128, 8), 0)
    k = lax.broadcasted_iota(jnp.int32, (128, 8), 1)
    e = jnp.where(j // DIM == k, 1.0, 0.0).astype(jnp.float32)
    o_ref[...] = lax.dot_general(e, xw, (((0,), (1,)), ((), ())),
                                 preferred_element_type=jnp.float32)


def _sc_body(items_hbm, user_hbm, out_hbm,
             in0, in1, ou0, ou1, u_v,
             semh0, semh1, semo0, semo1):
    cid = lax.axis_index("c")
    sid = lax.axis_index("s")
    wid = sid * NUM_CORES + cid

    pltpu.sync_copy(user_hbm, u_v)
    u_vec = u_v[0, :]
    u = [u_vec[d] for d in range(DIM)]

    lanes = lax.iota(jnp.int32, LANES)
    idx = [lanes * DIM + d for d in range(DIM)]

    ins = (in0, in1)
    outs = (ou0, ou1)
    semh = (semh0, semh1)
    semo = (semo0, semo1)

    def sub_slices(c, b):
        base = pl.multiple_of(c * (CHUNK * DIM), CHUNK * DIM)
        for q in range(KSTREAMS):
            yield (items_hbm.at[pl.ds(base + q * SUBW, SUBW)],
                   ins[b].at[pl.ds(q * SUBW, SUBW)])

    def start_h(c, b):
        for src, dst in sub_slices(c, b):
            pltpu.async_copy(src, dst, semh[b])

    def wait_h(c, b):
        for src, dst in sub_slices(c, b):
            pltpu.make_async_copy(src, dst, semh[b]).wait()

    def out_slice(c):
        row0 = pl.multiple_of(c * CHUNK, CHUNK)
        return out_hbm.at[pl.ds(row0, CHUNK)]

    def start_o(c, b):
        pltpu.async_copy(outs[b], out_slice(c), semo[b])

    def wait_o(c, b):
        pltpu.make_async_copy(outs[b], out_slice(c), semo[b]).wait()

    def compute(b):
        in_v = ins[b]
        out_v = outs[b]

        def grp(g, carry):
            base = g * (LANES * DIM)
            ix = [idx[d] + base for d in range(DIM)]
            acc0 = u[0] * plsc.load_gather(in_v, [ix[0]])
            acc1 = u[1] * plsc.load_gather(in_v, [ix[1]])
            acc2 = u[2] * plsc.load_gather(in_v, [ix[2]])
            acc3 = u[3] * plsc.load_gather(in_v, [ix[3]])
            for d in range(4, DIM, 4):
                acc0 += u[d] * plsc.load_gather(in_v, [ix[d]])
                acc1 += u[d + 1] * plsc.load_gather(in_v, [ix[d + 1]])
                acc2 += u[d + 2] * plsc.load_gather(in_v, [ix[d + 2]])
                acc3 += u[d + 3] * plsc.load_gather(in_v, [ix[d + 3]])
            out_v[pl.ds(g * LANES, LANES)] = (acc0 + acc1) + (acc2 + acc3)
            return carry

        lax.fori_loop(0, GROUPS, grp, 0)

    c0 = wid
    c1 = wid + NUM_WORKERS
    c2 = wid + 2 * NUM_WORKERS
    has3 = c2 < NSC_CHUNKS

    start_h(c0, 0)
    start_h(c1, 1)

    wait_h(c0, 0)
    compute(0)
    start_o(c0, 0)

    @pl.when(has3)
    def _():
        start_h(c2, 0)

    wait_h(c1, 1)
    compute(1)
    start_o(c1, 1)

    @pl.when(has3)
    def _():
        wait_h(c2, 0)
        wait_o(c0, 0)
        compute(0)
        start_o(c2, 0)
        wait_o(c2, 0)

    @pl.when(jnp.logical_not(has3))
    def _():
        wait_o(c0, 0)

    wait_o(c1, 1)


@functools.partial(jax.jit, static_argnames=())
def kernel(items_emb, user_emb):
    x2 = items_emb.reshape(M_ITEMS * DIM // 128, 128)
    u128 = jnp.tile(user_emb, (1, 8))
    tc_out = pl.pallas_call(
        _tc_body,
        grid=(TC_GRID,),
        in_specs=[
            pl.BlockSpec((1, 128), lambda i: (0, 0)),
            pl.BlockSpec((RC, 128), lambda i: (i, 0)),
        ],
        out_specs=pl.BlockSpec((8, RC), lambda i: (0, i)),
        out_shape=jax.ShapeDtypeStruct((8, M_ITEMS // 8), jnp.float32),
    )(u128, x2)

    sc_in = items_emb[N_TC:].reshape(N_SC * DIM)
    mesh = plsc.VectorSubcoreMesh(
        core_axis_name="c", subcore_axis_name="s",
        num_cores=NUM_CORES, num_subcores=NUM_SUBCORES,
    )
    run = pl.kernel(
        _sc_body,
        out_type=jax.ShapeDtypeStruct((N_SC,), jnp.float32),
        mesh=mesh,
        scratch_types=[
            pltpu.VMEM((CHUNK * DIM,), jnp.float32),
            pltpu.VMEM((CHUNK * DIM,), jnp.float32),
            pltpu.VMEM((CHUNK,), jnp.float32),
            pltpu.VMEM((CHUNK,), jnp.float32),
            pltpu.VMEM((1, DIM), jnp.float32),
            pltpu.SemaphoreType.DMA,
            pltpu.SemaphoreType.DMA,
            pltpu.SemaphoreType.DMA,
            pltpu.SemaphoreType.DMA,
        ],
        compiler_params=pltpu.CompilerParams(needs_layout_passes=False),
    )
    sc_out = run(sc_in, user_emb)

    tc_scores = tc_out.T.reshape(M_ITEMS)
    return lax.dynamic_update_slice(tc_scores, sc_out, (N_TC,))


# speedup vs baseline: 1.0784x; 1.0784x over previous
import functools

import jax
import jax.numpy as jnp
from jax import lax
from jax.experimental import pallas as pl
from jax.experimental.pallas import tpu as pltpu
from jax.experimental.pallas import tpu_sc as plsc

M_ITEMS = 1_000_000
DIM = 16
LANES = 16
NUM_CORES = 2
NUM_SUBCORES = 16
NUM_WORKERS = NUM_CORES * NUM_SUBCORES
CHUNK = 1600
NCHUNKS = M_ITEMS // CHUNK
GROUPS = CHUNK // LANES
PAIRS = -(-NCHUNKS // NUM_WORKERS) // 2
KSTREAMS = 8
SUBW = CHUNK * DIM // KSTREAMS


def _sc_body(items_hbm, user_hbm, out_hbm,
             in0, in1, ou0, ou1, u_v,
             semh0, semh1, semo0, semo1):
    cid = lax.axis_index("c")
    sid = lax.axis_index("s")
    wid = sid * NUM_CORES + cid

    pltpu.sync_copy(user_hbm, u_v)
    u_vec = u_v[0, :]
    u = [u_vec[d] for d in range(DIM)]

    lanes = lax.iota(jnp.int32, LANES)
    idx = [lanes * DIM + d for d in range(DIM)]

    ins = (in0, in1)
    outs = (ou0, ou1)
    semh = (semh0, semh1)
    semo = (semo0, semo1)

    def sub_slices(c, b):
        base = pl.multiple_of(c * (CHUNK * DIM), CHUNK * DIM)
        for q in range(KSTREAMS):
            yield (items_hbm.at[pl.ds(base + q * SUBW, SUBW)],
                   ins[b].at[pl.ds(q * SUBW, SUBW)])

    def start_h(c, b):
        for src, dst in sub_slices(c, b):
            pltpu.async_copy(src, dst, semh[b])

    def wait_h(c, b):
        for src, dst in sub_slices(c, b):
            pltpu.make_async_copy(src, dst, semh[b]).wait()

    def out_slice(c):
        row0 = pl.multiple_of(c * CHUNK, CHUNK)
        return out_hbm.at[pl.ds(row0, CHUNK)]

    def start_o(c, b):
        pltpu.async_copy(outs[b], out_slice(c), semo[b])

    def wait_o(c, b):
        pltpu.make_async_copy(outs[b], out_slice(c), semo[b]).wait()

    def compute(b):
        in_v = ins[b]
        out_v = outs[b]

        def grp(g, carry):
            base = g * (LANES * DIM)
            ix = [idx[d] + base for d in range(DIM)]
            acc0 = u[0] * plsc.load_gather(in_v, [ix[0]])
            acc1 = u[1] * plsc.load_gather(in_v, [ix[1]])
            acc2 = u[2] * plsc.load_gather(in_v, [ix[2]])
            acc3 = u[3] * plsc.load_gather(in_v, [ix[3]])
            for d in range(4, DIM, 4):
                acc0 += u[d] * plsc.load_gather(in_v, [ix[d]])
                acc1 += u[d + 1] * plsc.load_gather(in_v, [ix[d + 1]])
                acc2 += u[d + 2] * plsc.load_gather(in_v, [ix[d + 2]])
                acc3 += u[d + 3] * plsc.load_gather(in_v, [ix[d + 3]])
            out_v[pl.ds(g * LANES, LANES)] = (acc0 + acc1) + (acc2 + acc3)
            return carry

        lax.fori_loop(0, GROUPS, grp, 0)

    def sub_iter(j, off):
        b = off
        nb = 1 - off
        c = wid + 64 * j + 32 * off

        @pl.when(c < NCHUNKS)
        def _():
            wait_h(c, b)

            @pl.when(c + NUM_WORKERS < NCHUNKS)
            def _():
                start_h(c + NUM_WORKERS, nb)

            @pl.when(2 * j + off >= 2)
            def _():
                wait_o(c - 2 * NUM_WORKERS, b)

            compute(b)
            start_o(c, b)

    start_h(wid, 0)

    def pair_body(j, carry):
        sub_iter(j, 0)
        sub_iter(j, 1)
        return carry

    lax.fori_loop(0, PAIRS, pair_body, 0)

    wait_o(wid, 0)
    wait_o(wid, 1)


@functools.partial(jax.jit, static_argnames=())
def kernel(items_emb, user_emb):
    mesh = plsc.VectorSubcoreMesh(
        core_axis_name="c", subcore_axis_name="s",
        num_cores=NUM_CORES, num_subcores=NUM_SUBCORES,
    )
    run = pl.kernel(
        _sc_body,
        out_type=jax.ShapeDtypeStruct((M_ITEMS,), jnp.float32),
        mesh=mesh,
        scratch_types=[
            pltpu.VMEM((CHUNK * DIM,), jnp.float32),
            pltpu.VMEM((CHUNK * DIM,), jnp.float32),
            pltpu.VMEM((CHUNK,), jnp.float32),
            pltpu.VMEM((CHUNK,), jnp.float32),
            pltpu.VMEM((1, DIM), jnp.float32),
            pltpu.SemaphoreType.DMA,
            pltpu.SemaphoreType.DMA,
            pltpu.SemaphoreType.DMA,
            pltpu.SemaphoreType.DMA,
        ],
        compiler_params=pltpu.CompilerParams(needs_layout_passes=False),
    )
    return run(items_emb.reshape(M_ITEMS * DIM), user_emb)
